# Initial kernel scaffold; baseline (speedup 1.0000x reference)
import jax
import jax.numpy as jnp
from jax.experimental import pallas as pl


def kernel(x, edge_index, batch, params):
    raise NotImplementedError("write your pallas kernel here")


# speedup vs baseline: 3.2159x; 3.2159x over previous
import functools

import jax
import jax.numpy as jnp
from jax import lax
from jax.experimental import pallas as pl
from jax.experimental.pallas import tpu as pltpu
from jax.experimental.pallas import tpu_sc as plsc

N = 10000
E = 320000
D = 128
B = 64
NOUT = 1

NC = 2
NS = 16
NW = NC * NS
NPW = 320
NPAD = NW * NPW
C = 128
EPAD = E + 2 * C
GPW = B // NW
R = 256

_Z16 = lambda: jnp.zeros((16,), jnp.float32)
_FLOOR = -3.0e38
_F16 = lambda: jnp.full((16,), _FLOOR, jnp.float32)


def _agg_body(h_hbm, src_hbm, dst_hbm, rp_hbm,
              sum_hbm, max_hbm, cnt_hbm,
              rows_v, idx_v, dstc_v, rp_v,
              osum_v, omax_v, ocnt_v, sem):
    wid = lax.axis_index("s") * NC + lax.axis_index("c")
    n0 = wid * NPW

    def _zi(i, _):
        r = i // 8
        c = (i % 8) * 16
        osum_v[r, pl.ds(c, 16)] = _Z16()
        omax_v[r, pl.ds(c, 16)] = _Z16()
        return 0
    lax.fori_loop(0, NPW * 8, _zi, 0, unroll=8)

    pltpu.sync_copy(rp_hbm.at[pl.ds(n0, NPW + 16)], rp_v)
    for i in range(NPW // 16):
        a = rp_v[pl.ds(i * 16, 16)]
        b = rp_v[pl.ds(i * 16 + 1, 16)]
        ocnt_v[pl.ds(i * 16, 16)] = (b - a).astype(jnp.float32)
    e0 = rp_v[pl.ds(0, 16)][0]
    e1 = rp_v[pl.ds(NPW, 16)][0]
    ae0 = (e0 // 8) * 8
    nch = (e1 - ae0 + (C - 1)) // C

    def _chunk(k, carry):
        base = ae0 + k * C
        pltpu.sync_copy(src_hbm.at[pl.ds(base, C)], idx_v)
        pltpu.sync_copy(dst_hbm.at[pl.ds(base, C + 24)], dstc_v)
        pltpu.async_copy(h_hbm.at[idx_v], rows_v, sem).wait()

        def _edge(le, ec):
            accs = ec[0:8]
            accm = ec[8:16]
            rows = [rows_v[le, pl.ds(16 * c, 16)] for c in range(8)]
            accs = [a + r for a, r in zip(accs, rows)]
            accm = [jnp.maximum(a, r) for a, r in zip(accm, rows)]
            d2 = dstc_v[pl.ds(le, 16)]
            d = d2[0]
            dn = d2[1]
            dloc = d - n0
            flush = d != dn
            inb = (dloc >= 0) & (dloc < NPW)

            @pl.when(flush & inb)
            def _():
                for c in range(8):
                    osum_v[dloc, pl.ds(16 * c, 16)] = accs[c]
                    omax_v[dloc, pl.ds(16 * c, 16)] = accm[c]

            kf = jnp.full((16,), flush.astype(jnp.float32))
            zf = 1.0 - kf
            kff = kf * _FLOOR
            accs = [a * zf for a in accs]
            accm = [a * zf + kff for a in accm]
            return tuple(accs) + tuple(accm)

        return lax.fori_loop(0, C, _edge, carry)

    init = tuple(_Z16() for _ in range(8)) + tuple(_F16() for _ in range(8))
    lax.fori_loop(0, nch, _chunk, init)

    pltpu.sync_copy(osum_v, sum_hbm.at[pl.ds(n0, NPW)])
    pltpu.sync_copy(omax_v, max_hbm.at[pl.ds(n0, NPW)])
    pltpu.sync_copy(ocnt_v, cnt_hbm.at[pl.ds(n0, NPW)])


@functools.lru_cache(maxsize=None)
def _agg_call():
    mesh = plsc.VectorSubcoreMesh(
        core_axis_name="c", subcore_axis_name="s", num_cores=NC, num_subcores=NS)
    return pl.kernel(
        _agg_body,
        out_type=(jax.ShapeDtypeStruct((NPAD, D), jnp.float32),
                  jax.ShapeDtypeStruct((NPAD, D), jnp.float32),
                  jax.ShapeDtypeStruct((NPAD,), jnp.float32)),
        mesh=mesh,
        scratch_types=(pltpu.VMEM((C, D), jnp.float32),
                       pltpu.VMEM((C,), jnp.int32),
                       pltpu.VMEM((C + 24,), jnp.int32),
                       pltpu.VMEM((NPW + 16,), jnp.int32),
                       pltpu.VMEM((NPW, D), jnp.float32),
                       pltpu.VMEM((NPW, D), jnp.float32),
                       pltpu.VMEM((NPW,), jnp.float32),
                       pltpu.SemaphoreType.DMA))


def _pool_body(h_hbm, bp_hbm, sum_hbm, max_hbm, mean_hbm,
               rows_v, bp_v, osum_v, omax_v, omean_v):
    wid = lax.axis_index("s") * NC + lax.axis_index("c")
    pltpu.sync_copy(bp_hbm.at[pl.ds(0, 80)], bp_v)

    for j in range(GPW):
        g = wid * GPW + j
        b2 = bp_v[pl.ds(g, 16)]
        r0 = b2[0]
        r1 = b2[1]
        ar0 = (r0 // 8) * 8
        nch = (r1 - ar0 + (C - 1)) // C

        def _chunk(k, carry, j=j, r0=r0, r1=r1, ar0=ar0):
            base = ar0 + k * C
            pltpu.sync_copy(h_hbm.at[pl.ds(base, C)], rows_v)

            def _row(le, rc):
                gr = base + le
                ok = ((gr >= r0) & (gr < r1)).astype(jnp.float32)
                okv = jnp.full((16,), ok)
                nokf = (1.0 - okv) * _FLOOR
                rows = [rows_v[le, pl.ds(16 * c, 16)] for c in range(8)]
                return tuple(a + r * okv for a, r in zip(rc[0:8], rows)) + \
                       tuple(jnp.maximum(a, r * okv + nokf)
                             for a, r in zip(rc[8:16], rows))

            return lax.fori_loop(0, C, _row, carry)

        init = tuple(_Z16() for _ in range(8)) + tuple(_F16() for _ in range(8))
        fin = lax.fori_loop(0, nch, _chunk, init)

        cntf = (r1 - r0).astype(jnp.float32)
        cv = jnp.full((16,), cntf)
        inv = 1.0 / jnp.maximum(cv, 1.0)
        nzv = jnp.minimum(cv, 1.0)
        for c in range(8):
            s = fin[c]
            m = fin[8 + c] * nzv
            osum_v[j, pl.ds(16 * c, 16)] = s
            omax_v[j, pl.ds(16 * c, 16)] = m
            omean_v[j, pl.ds(16 * c, 16)] = s * inv

    g0 = wid * GPW
    pltpu.sync_copy(osum_v, sum_hbm.at[pl.ds(g0, GPW)])
    pltpu.sync_copy(omax_v, max_hbm.at[pl.ds(g0, GPW)])
    pltpu.sync_copy(omean_v, mean_hbm.at[pl.ds(g0, GPW)])


@functools.lru_cache(maxsize=None)
def _pool_call():
    mesh = plsc.VectorSubcoreMesh(
        core_axis_name="c", subcore_axis_name="s", num_cores=NC, num_subcores=NS)
    return pl.kernel(
        _pool_body,
        out_type=(jax.ShapeDtypeStruct((B, D), jnp.float32),
                  jax.ShapeDtypeStruct((B, D), jnp.float32),
                  jax.ShapeDtypeStruct((B, D), jnp.float32)),
        mesh=mesh,
        scratch_types=(pltpu.VMEM((C, D), jnp.float32),
                       pltpu.VMEM((80,), jnp.int32),
                       pltpu.VMEM((GPW, D), jnp.float32),
                       pltpu.VMEM((GPW, D), jnp.float32),
                       pltpu.VMEM((GPW, D), jnp.float32)))


def _silu(v):
    return v * (1.0 / (1.0 + jnp.exp(-v)))


def _sage_tc(x_ref, s_ref, m_ref, cnt_ref, wl_ref, wr_ref, bl_ref, o_ref):
    s = s_ref[...]
    inv = 1.0 / jnp.maximum(cnt_ref[...], 1.0)
    agg = jnp.concatenate([m_ref[...], s, s * inv], axis=1)
    y = (jnp.dot(agg, wl_ref[...], preferred_element_type=jnp.float32)
         + jnp.dot(x_ref[...], wr_ref[...], preferred_element_type=jnp.float32)
         + bl_ref[...])
    o_ref[...] = _silu(y)


def _mlp_tc(h_ref, w1_ref, b1_ref, g1_ref, be1_ref, w2_ref, b2_ref, o_ref):
    t = jnp.dot(h_ref[...], w1_ref[...], preferred_element_type=jnp.float32) + b1_ref[...]
    t = _silu(t)
    mu = t.mean(-1, keepdims=True)
    var = ((t - mu) ** 2).mean(-1, keepdims=True)
    t = (t - mu) * lax.rsqrt(var + 1e-5) * g1_ref[...] + be1_ref[...]
    o_ref[...] = jnp.dot(t, w2_ref[...], preferred_element_type=jnp.float32) + b2_ref[...]


def _ro_tc(mn_ref, mx_ref, sm_ref, w1_ref, b1_ref, g1_ref, be1_ref, w2_ref, b2_ref, o_ref):
    out = jnp.concatenate([mn_ref[...], mx_ref[...], sm_ref[...]], axis=1)
    t = jnp.dot(out, w1_ref[...], preferred_element_type=jnp.float32) + b1_ref[...]
    t = _silu(t)
    mu = t.mean(-1, keepdims=True)
    var = ((t - mu) ** 2).mean(-1, keepdims=True)
    t = (t - mu) * lax.rsqrt(var + 1e-5) * g1_ref[...] + be1_ref[...]
    o_ref[...] = jnp.dot(t, w2_ref[...], preferred_element_type=jnp.float32) + b2_ref[...]


def _row_block(shape):
    return pl.BlockSpec(shape, lambda i: (i, 0))


def _whole(shape):
    return pl.BlockSpec(shape, lambda i: (0, 0))


_GRID = NPAD // R

_sage_call = pl.pallas_call(
    _sage_tc,
    grid=(_GRID,),
    in_specs=[_row_block((R, D)), _row_block((R, D)), _row_block((R, D)),
              _row_block((R, 1)),
              _whole((3 * D, D)), _whole((D, D)), _whole((1, D))],
    out_specs=_row_block((R, D)),
    out_shape=jax.ShapeDtypeStruct((NPAD, D), jnp.float32),
)

_mlp_call = pl.pallas_call(
    _mlp_tc,
    grid=(_GRID,),
    in_specs=[_row_block((R, D)),
              _whole((D, 4 * D)), _whole((1, 4 * D)), _whole((1, 4 * D)),
              _whole((1, 4 * D)), _whole((4 * D, D)), _whole((1, D))],
    out_specs=_row_block((R, D)),
    out_shape=jax.ShapeDtypeStruct((NPAD, D), jnp.float32),
)

_ro_call = pl.pallas_call(
    _ro_tc,
    in_specs=[pl.BlockSpec((B, D)), pl.BlockSpec((B, D)), pl.BlockSpec((B, D)),
              pl.BlockSpec((3 * D, 4 * D)), pl.BlockSpec((1, 4 * D)),
              pl.BlockSpec((1, 4 * D)), pl.BlockSpec((1, 4 * D)),
              pl.BlockSpec((4 * D, D)), pl.BlockSpec((1, D))],
    out_specs=pl.BlockSpec((B, D)),
    out_shape=jax.ShapeDtypeStruct((B, D), jnp.float32),
)


def kernel(x, edge_index, batch, params):
    src = edge_index[0]
    dst = edge_index[1]
    dst_s, src_s = lax.sort([dst, src], num_keys=1)
    rp = jnp.searchsorted(
        dst_s, jnp.arange(NPAD + NPW + 16, dtype=jnp.int32),
        side="left").astype(jnp.int32)
    src_p = jnp.concatenate([src_s, jnp.zeros((EPAD - E,), jnp.int32)])
    dst_p = jnp.concatenate([dst_s, jnp.full((EPAD - E,), NPAD, jnp.int32)])
    bp = jnp.searchsorted(
        batch, jnp.arange(80, dtype=jnp.int32), side="left").astype(jnp.int32)

    h = jnp.zeros((NPAD, D), jnp.float32).at[:N].set(x)
    for p in params["convs"]:
        ssum, smax, scnt = _agg_call()(h, src_p, dst_p, rp)
        h = _sage_call(h, ssum, smax, scnt.reshape(NPAD, 1),
                       p["Wl"], p["Wr"], p["bl"].reshape(1, D))

    m = params["mlp"]
    h = _mlp_call(h, m["W1"], m["b1"].reshape(1, 4 * D), m["g1"].reshape(1, 4 * D),
                  m["be1"].reshape(1, 4 * D), m["W2"], m["b2"].reshape(1, D))

    psum, pmax, pmean = _pool_call()(h, bp)

    r = params["ro"]
    w2p = jnp.zeros((4 * D, D), jnp.float32).at[:, :2 * NOUT].set(r["W2"])
    b2p = jnp.zeros((1, D), jnp.float32).at[0, :2 * NOUT].set(r["b2"])
    out = _ro_call(pmean, pmax, psum,
                   r["W1"], r["b1"].reshape(1, 4 * D), r["g1"].reshape(1, 4 * D),
                   r["be1"].reshape(1, 4 * D), w2p, b2p)
    return out[:, :2 * NOUT]


# speedup vs baseline: 4.0034x; 1.2449x over previous
import functools

import jax
import jax.numpy as jnp
from jax import lax
from jax.experimental import pallas as pl
from jax.experimental.pallas import tpu as pltpu
from jax.experimental.pallas import tpu_sc as plsc

N = 10000
E = 320000
D = 128
B = 64
NOUT = 1

NC = 2
NS = 16
NW = NC * NS
NPW = 320
NPAD = NW * NPW
C = 128
EPAD = E + 2 * C
GPW = B // NW
R = 256

_Z16 = lambda: jnp.zeros((16,), jnp.float32)
_FLOOR = -3.0e38
_F16 = lambda: jnp.full((16,), _FLOOR, jnp.float32)


def _agg_body(h_hbm, src_hbm, dst_hbm, rp_hbm,
              sum_hbm, max_hbm, cnt_hbm,
              rows_v, idx_v, dstc_v, rp_v,
              osum_v, omax_v, ocnt_v, sem):
    wid = lax.axis_index("s") * NC + lax.axis_index("c")
    n0 = wid * NPW

    def _zi(i, _):
        r = i // 8
        c = (i % 8) * 16
        osum_v[r, pl.ds(c, 16)] = _Z16()
        omax_v[r, pl.ds(c, 16)] = _Z16()
        return 0
    lax.fori_loop(0, NPW * 8, _zi, 0, unroll=8)

    pltpu.sync_copy(rp_hbm.at[pl.ds(n0, NPW + 16)], rp_v)
    for i in range(NPW // 16):
        a = rp_v[pl.ds(i * 16, 16)]
        b = rp_v[pl.ds(i * 16 + 1, 16)]
        ocnt_v[pl.ds(i * 16, 16)] = (b - a).astype(jnp.float32)
    e0 = rp_v[pl.ds(0, 16)][0]
    e1 = rp_v[pl.ds(NPW, 16)][0]
    ae0 = (e0 // 8) * 8
    nch = (e1 - ae0 + (C - 1)) // C

    def _chunk(k, carry):
        base = ae0 + k * C
        pltpu.sync_copy(src_hbm.at[pl.ds(base, C)], idx_v)
        pltpu.sync_copy(dst_hbm.at[pl.ds(base, C + 24)], dstc_v)
        pltpu.async_copy(h_hbm.at[idx_v], rows_v, sem).wait()

        def _edge(le, ec):
            accs = ec[0:8]
            accm = ec[8:16]
            d2 = dstc_v[pl.ds(le, 16)]
            d = d2[0]
            dn = d2[1]
            dloc = d - n0
            flush = d != dn
            inb = (dloc >= 0) & (dloc < NPW)
            rows = [rows_v[le, pl.ds(16 * c, 16)] for c in range(8)]
            accs = [a + r for a, r in zip(accs, rows)]
            accm = [jnp.maximum(a, r) for a, r in zip(accm, rows)]

            @pl.when(flush & inb)
            def _():
                for c in range(8):
                    osum_v[dloc, pl.ds(16 * c, 16)] = accs[c]
                    omax_v[dloc, pl.ds(16 * c, 16)] = accm[c]

            kf = jnp.full((16,), flush.astype(jnp.float32))
            zf = 1.0 - kf
            kff = kf * _FLOOR
            accs = [a * zf for a in accs]
            accm = [a * zf + kff for a in accm]
            return tuple(accs) + tuple(accm)

        return lax.fori_loop(0, C, _edge, carry, unroll=4)

    init = tuple(_Z16() for _ in range(8)) + tuple(_F16() for _ in range(8))
    lax.fori_loop(0, nch, _chunk, init)

    pltpu.sync_copy(osum_v, sum_hbm.at[pl.ds(n0, NPW)])
    pltpu.sync_copy(omax_v, max_hbm.at[pl.ds(n0, NPW)])
    pltpu.sync_copy(ocnt_v, cnt_hbm.at[pl.ds(n0, NPW)])


@functools.lru_cache(maxsize=None)
def _agg_call():
    mesh = plsc.VectorSubcoreMesh(
        core_axis_name="c", subcore_axis_name="s", num_cores=NC, num_subcores=NS)
    return pl.kernel(
        _agg_body,
        out_type=(jax.ShapeDtypeStruct((NPAD, D), jnp.float32),
                  jax.ShapeDtypeStruct((NPAD, D), jnp.float32),
                  jax.ShapeDtypeStruct((NPAD,), jnp.float32)),
        mesh=mesh,
        scratch_types=(pltpu.VMEM((C, D), jnp.float32),
                       pltpu.VMEM((C,), jnp.int32),
                       pltpu.VMEM((C + 24,), jnp.int32),
                       pltpu.VMEM((NPW + 16,), jnp.int32),
                       pltpu.VMEM((NPW, D), jnp.float32),
                       pltpu.VMEM((NPW, D), jnp.float32),
                       pltpu.VMEM((NPW,), jnp.float32),
                       pltpu.SemaphoreType.DMA))


def _pool_body(h_hbm, bp_hbm, sum_hbm, max_hbm, mean_hbm,
               rows_v, bp_v, osum_v, omax_v, omean_v):
    wid = lax.axis_index("s") * NC + lax.axis_index("c")
    pltpu.sync_copy(bp_hbm.at[pl.ds(0, 80)], bp_v)

    for j in range(GPW):
        g = wid * GPW + j
        b2 = bp_v[pl.ds(g, 16)]
        r0 = b2[0]
        r1 = b2[1]
        ar0 = (r0 // 8) * 8
        nch = (r1 - ar0 + (C - 1)) // C

        def _chunk(k, carry, j=j, r0=r0, r1=r1, ar0=ar0):
            base = ar0 + k * C
            pltpu.sync_copy(h_hbm.at[pl.ds(base, C)], rows_v)

            def _row(le, rc):
                gr = base + le
                ok = ((gr >= r0) & (gr < r1)).astype(jnp.float32)
                okv = jnp.full((16,), ok)
                nokf = (1.0 - okv) * _FLOOR
                rows = [rows_v[le, pl.ds(16 * c, 16)] for c in range(8)]
                return tuple(a + r * okv for a, r in zip(rc[0:8], rows)) + \
                       tuple(jnp.maximum(a, r * okv + nokf)
                             for a, r in zip(rc[8:16], rows))

            return lax.fori_loop(0, C, _row, carry)

        init = tuple(_Z16() for _ in range(8)) + tuple(_F16() for _ in range(8))
        fin = lax.fori_loop(0, nch, _chunk, init)

        cntf = (r1 - r0).astype(jnp.float32)
        cv = jnp.full((16,), cntf)
        inv = 1.0 / jnp.maximum(cv, 1.0)
        nzv = jnp.minimum(cv, 1.0)
        for c in range(8):
            s = fin[c]
            m = fin[8 + c] * nzv
            osum_v[j, pl.ds(16 * c, 16)] = s
            omax_v[j, pl.ds(16 * c, 16)] = m
            omean_v[j, pl.ds(16 * c, 16)] = s * inv

    g0 = wid * GPW
    pltpu.sync_copy(osum_v, sum_hbm.at[pl.ds(g0, GPW)])
    pltpu.sync_copy(omax_v, max_hbm.at[pl.ds(g0, GPW)])
    pltpu.sync_copy(omean_v, mean_hbm.at[pl.ds(g0, GPW)])


@functools.lru_cache(maxsize=None)
def _pool_call():
    mesh = plsc.VectorSubcoreMesh(
        core_axis_name="c", subcore_axis_name="s", num_cores=NC, num_subcores=NS)
    return pl.kernel(
        _pool_body,
        out_type=(jax.ShapeDtypeStruct((B, D), jnp.float32),
                  jax.ShapeDtypeStruct((B, D), jnp.float32),
                  jax.ShapeDtypeStruct((B, D), jnp.float32)),
        mesh=mesh,
        scratch_types=(pltpu.VMEM((C, D), jnp.float32),
                       pltpu.VMEM((80,), jnp.int32),
                       pltpu.VMEM((GPW, D), jnp.float32),
                       pltpu.VMEM((GPW, D), jnp.float32),
                       pltpu.VMEM((GPW, D), jnp.float32)))


def _silu(v):
    return v * (1.0 / (1.0 + jnp.exp(-v)))


def _sage_tc(x_ref, s_ref, m_ref, cnt_ref, wl_ref, wr_ref, bl_ref, o_ref):
    s = s_ref[...]
    inv = 1.0 / jnp.maximum(cnt_ref[...], 1.0)
    agg = jnp.concatenate([m_ref[...], s, s * inv], axis=1)
    y = (jnp.dot(agg, wl_ref[...], preferred_element_type=jnp.float32)
         + jnp.dot(x_ref[...], wr_ref[...], preferred_element_type=jnp.float32)
         + bl_ref[...])
    o_ref[...] = _silu(y)


def _mlp_tc(h_ref, w1_ref, b1_ref, g1_ref, be1_ref, w2_ref, b2_ref, o_ref):
    t = jnp.dot(h_ref[...], w1_ref[...], preferred_element_type=jnp.float32) + b1_ref[...]
    t = _silu(t)
    mu = t.mean(-1, keepdims=True)
    var = ((t - mu) ** 2).mean(-1, keepdims=True)
    t = (t - mu) * lax.rsqrt(var + 1e-5) * g1_ref[...] + be1_ref[...]
    o_ref[...] = jnp.dot(t, w2_ref[...], preferred_element_type=jnp.float32) + b2_ref[...]


def _ro_tc(mn_ref, mx_ref, sm_ref, w1_ref, b1_ref, g1_ref, be1_ref, w2_ref, b2_ref, o_ref):
    out = jnp.concatenate([mn_ref[...], mx_ref[...], sm_ref[...]], axis=1)
    t = jnp.dot(out, w1_ref[...], preferred_element_type=jnp.float32) + b1_ref[...]
    t = _silu(t)
    mu = t.mean(-1, keepdims=True)
    var = ((t - mu) ** 2).mean(-1, keepdims=True)
    t = (t - mu) * lax.rsqrt(var + 1e-5) * g1_ref[...] + be1_ref[...]
    o_ref[...] = jnp.dot(t, w2_ref[...], preferred_element_type=jnp.float32) + b2_ref[...]


def _row_block(shape):
    return pl.BlockSpec(shape, lambda i: (i, 0))


def _whole(shape):
    return pl.BlockSpec(shape, lambda i: (0, 0))


_GRID = NPAD // R

_sage_call = pl.pallas_call(
    _sage_tc,
    grid=(_GRID,),
    in_specs=[_row_block((R, D)), _row_block((R, D)), _row_block((R, D)),
              _row_block((R, 1)),
              _whole((3 * D, D)), _whole((D, D)), _whole((1, D))],
    out_specs=_row_block((R, D)),
    out_shape=jax.ShapeDtypeStruct((NPAD, D), jnp.float32),
)

_mlp_call = pl.pallas_call(
    _mlp_tc,
    grid=(_GRID,),
    in_specs=[_row_block((R, D)),
              _whole((D, 4 * D)), _whole((1, 4 * D)), _whole((1, 4 * D)),
              _whole((1, 4 * D)), _whole((4 * D, D)), _whole((1, D))],
    out_specs=_row_block((R, D)),
    out_shape=jax.ShapeDtypeStruct((NPAD, D), jnp.float32),
)

_ro_call = pl.pallas_call(
    _ro_tc,
    in_specs=[pl.BlockSpec((B, D)), pl.BlockSpec((B, D)), pl.BlockSpec((B, D)),
              pl.BlockSpec((3 * D, 4 * D)), pl.BlockSpec((1, 4 * D)),
              pl.BlockSpec((1, 4 * D)), pl.BlockSpec((1, 4 * D)),
              pl.BlockSpec((4 * D, D)), pl.BlockSpec((1, D))],
    out_specs=pl.BlockSpec((B, D)),
    out_shape=jax.ShapeDtypeStruct((B, D), jnp.float32),
)


def kernel(x, edge_index, batch, params):
    src = edge_index[0]
    dst = edge_index[1]
    dst_s, src_s = lax.sort([dst, src], num_keys=1)
    counts = jnp.zeros((NPAD + NPW + 15,), jnp.int32).at[dst].add(1)
    rp = jnp.concatenate([jnp.zeros((1,), jnp.int32),
                          jnp.cumsum(counts, dtype=jnp.int32)])
    src_p = jnp.concatenate([src_s, jnp.zeros((EPAD - E,), jnp.int32)])
    dst_p = jnp.concatenate([dst_s, jnp.full((EPAD - E,), NPAD, jnp.int32)])
    bp = jnp.searchsorted(
        batch, jnp.arange(80, dtype=jnp.int32), side="left").astype(jnp.int32)

    h = jnp.zeros((NPAD, D), jnp.float32).at[:N].set(x)
    for p in params["convs"]:
        ssum, smax, scnt = _agg_call()(h, src_p, dst_p, rp)
        h = _sage_call(h, ssum, smax, scnt.reshape(NPAD, 1),
                       p["Wl"], p["Wr"], p["bl"].reshape(1, D))

    m = params["mlp"]
    h = _mlp_call(h, m["W1"], m["b1"].reshape(1, 4 * D), m["g1"].reshape(1, 4 * D),
                  m["be1"].reshape(1, 4 * D), m["W2"], m["b2"].reshape(1, D))

    psum, pmax, pmean = _pool_call()(h, bp)

    r = params["ro"]
    w2p = jnp.zeros((4 * D, D), jnp.float32).at[:, :2 * NOUT].set(r["W2"])
    b2p = jnp.zeros((1, D), jnp.float32).at[0, :2 * NOUT].set(r["b2"])
    out = _ro_call(pmean, pmax, psum,
                   r["W1"], r["b1"].reshape(1, 4 * D), r["g1"].reshape(1, 4 * D),
                   r["be1"].reshape(1, 4 * D), w2p, b2p)
    return out[:, :2 * NOUT]


# speedup vs baseline: 4.8259x; 1.2054x over previous
import functools

import jax
import jax.numpy as jnp
from jax import lax
from jax.experimental import pallas as pl
from jax.experimental.pallas import tpu as pltpu
from jax.experimental.pallas import tpu_sc as plsc

N = 10000
E = 320000
D = 128
B = 64
NOUT = 1

NC = 2
NS = 16
NW = NC * NS
NPW = 320
NPAD = NW * NPW
C = 128
EPAD = E + 4 * C + 32
GPW = B // NW
R = 256

_Z16 = lambda: jnp.zeros((16,), jnp.float32)
_FLOOR = -3.0e38
_F16 = lambda: jnp.full((16,), _FLOOR, jnp.float32)


def _agg_body(h_hbm, src_hbm, dst_hbm, rp_hbm,
              sum_hbm, max_hbm, cnt_hbm,
              rows_v0, rows_v1, idx_v0, idx_v1, dstc_v0, dstc_v1, rp_v,
              osum_v, omax_v, ocnt_v, sem0, sem1):
    wid = lax.axis_index("s") * NC + lax.axis_index("c")
    n0 = wid * NPW

    def _zi(i, _):
        r = i // 8
        c = (i % 8) * 16
        osum_v[r, pl.ds(c, 16)] = _Z16()
        omax_v[r, pl.ds(c, 16)] = _Z16()
        return 0
    lax.fori_loop(0, NPW * 8, _zi, 0, unroll=8)

    pltpu.sync_copy(rp_hbm.at[pl.ds(n0, NPW + 16)], rp_v)
    for i in range(NPW // 16):
        a = rp_v[pl.ds(i * 16, 16)]
        b = rp_v[pl.ds(i * 16 + 1, 16)]
        ocnt_v[pl.ds(i * 16, 16)] = (b - a).astype(jnp.float32)
    e0 = rp_v[pl.ds(0, 16)][0]
    e1 = rp_v[pl.ds(NPW, 16)][0]
    ae0 = (e0 // 8) * 8
    nchp = (e1 - ae0 + (2 * C - 1)) // (2 * C)

    rows_b = (rows_v0, rows_v1)
    idx_b = (idx_v0, idx_v1)
    dst_b = (dstc_v0, dstc_v1)
    sem_b = (sem0, sem1)

    def _start(k, b):
        base = ae0 + k * C
        pltpu.sync_copy(src_hbm.at[pl.ds(base, C)], idx_b[b])
        pltpu.sync_copy(dst_hbm.at[pl.ds(base, C + 24)], dst_b[b])
        pltpu.async_copy(h_hbm.at[idx_b[b]], rows_b[b], sem_b[b])

    def _wait(b):
        pltpu.make_async_copy(h_hbm.at[idx_b[b]], rows_b[b], sem_b[b]).wait()

    def _compute(b, carry):
        def _edge(le, ec):
            accs = ec[0:8]
            accm = ec[8:16]
            d2 = dst_b[b][pl.ds(le, 16)]
            d = d2[0]
            dn = d2[1]
            dloc = d - n0
            flush = d != dn
            inb = (dloc >= 0) & (dloc < NPW)
            rows = [rows_b[b][le, pl.ds(16 * c, 16)] for c in range(8)]
            accs = [a + r for a, r in zip(accs, rows)]
            accm = [jnp.maximum(a, r) for a, r in zip(accm, rows)]

            @pl.when(flush & inb)
            def _():
                for c in range(8):
                    osum_v[dloc, pl.ds(16 * c, 16)] = accs[c]
                    omax_v[dloc, pl.ds(16 * c, 16)] = accm[c]

            kf = jnp.full((16,), flush.astype(jnp.float32))
            zf = 1.0 - kf
            kff = kf * _FLOOR
            accs = [a * zf for a in accs]
            accm = [a * zf + kff for a in accm]
            return tuple(accs) + tuple(accm)

        return lax.fori_loop(0, C, _edge, carry, unroll=4)

    _start(0, 0)

    def _pair(p, carry):
        k = 2 * p
        _start(k + 1, 1)
        _wait(0)
        carry = _compute(0, carry)
        _start(k + 2, 0)
        _wait(1)
        return _compute(1, carry)

    init = tuple(_Z16() for _ in range(8)) + tuple(_F16() for _ in range(8))
    lax.fori_loop(0, nchp, _pair, init)
    _wait(0)

    pltpu.sync_copy(osum_v, sum_hbm.at[pl.ds(n0, NPW)])
    pltpu.sync_copy(omax_v, max_hbm.at[pl.ds(n0, NPW)])
    pltpu.sync_copy(ocnt_v, cnt_hbm.at[pl.ds(n0, NPW)])


@functools.lru_cache(maxsize=None)
def _agg_call():
    mesh = plsc.VectorSubcoreMesh(
        core_axis_name="c", subcore_axis_name="s", num_cores=NC, num_subcores=NS)
    return pl.kernel(
        _agg_body,
        out_type=(jax.ShapeDtypeStruct((NPAD, D), jnp.float32),
                  jax.ShapeDtypeStruct((NPAD, D), jnp.float32),
                  jax.ShapeDtypeStruct((NPAD,), jnp.float32)),
        mesh=mesh,
        scratch_types=(pltpu.VMEM((C, D), jnp.float32),
                       pltpu.VMEM((C, D), jnp.float32),
                       pltpu.VMEM((C,), jnp.int32),
                       pltpu.VMEM((C,), jnp.int32),
                       pltpu.VMEM((C + 24,), jnp.int32),
                       pltpu.VMEM((C + 24,), jnp.int32),
                       pltpu.VMEM((NPW + 16,), jnp.int32),
                       pltpu.VMEM((NPW, D), jnp.float32),
                       pltpu.VMEM((NPW, D), jnp.float32),
                       pltpu.VMEM((NPW,), jnp.float32),
                       pltpu.SemaphoreType.DMA,
                       pltpu.SemaphoreType.DMA))


def _pool_body(h_hbm, bp_hbm, sum_hbm, max_hbm, mean_hbm,
               rows_v, bp_v, osum_v, omax_v, omean_v):
    wid = lax.axis_index("s") * NC + lax.axis_index("c")
    pltpu.sync_copy(bp_hbm.at[pl.ds(0, 80)], bp_v)

    for j in range(GPW):
        g = wid * GPW + j
        b2 = bp_v[pl.ds(g, 16)]
        r0 = b2[0]
        r1 = b2[1]
        ar0 = (r0 // 8) * 8
        nch = (r1 - ar0 + (C - 1)) // C

        def _chunk(k, carry, j=j, r0=r0, r1=r1, ar0=ar0):
            base = ar0 + k * C
            pltpu.sync_copy(h_hbm.at[pl.ds(base, C)], rows_v)

            def _row(le, rc):
                gr = base + le
                ok = ((gr >= r0) & (gr < r1)).astype(jnp.float32)
                okv = jnp.full((16,), ok)
                nokf = (1.0 - okv) * _FLOOR
                rows = [rows_v[le, pl.ds(16 * c, 16)] for c in range(8)]
                return tuple(a + r * okv for a, r in zip(rc[0:8], rows)) + \
                       tuple(jnp.maximum(a, r * okv + nokf)
                             for a, r in zip(rc[8:16], rows))

            return lax.fori_loop(0, C, _row, carry)

        init = tuple(_Z16() for _ in range(8)) + tuple(_F16() for _ in range(8))
        fin = lax.fori_loop(0, nch, _chunk, init)

        cntf = (r1 - r0).astype(jnp.float32)
        cv = jnp.full((16,), cntf)
        inv = 1.0 / jnp.maximum(cv, 1.0)
        nzv = jnp.minimum(cv, 1.0)
        for c in range(8):
            s = fin[c]
            m = fin[8 + c] * nzv
            osum_v[j, pl.ds(16 * c, 16)] = s
            omax_v[j, pl.ds(16 * c, 16)] = m
            omean_v[j, pl.ds(16 * c, 16)] = s * inv

    g0 = wid * GPW
    pltpu.sync_copy(osum_v, sum_hbm.at[pl.ds(g0, GPW)])
    pltpu.sync_copy(omax_v, max_hbm.at[pl.ds(g0, GPW)])
    pltpu.sync_copy(omean_v, mean_hbm.at[pl.ds(g0, GPW)])


@functools.lru_cache(maxsize=None)
def _pool_call():
    mesh = plsc.VectorSubcoreMesh(
        core_axis_name="c", subcore_axis_name="s", num_cores=NC, num_subcores=NS)
    return pl.kernel(
        _pool_body,
        out_type=(jax.ShapeDtypeStruct((B, D), jnp.float32),
                  jax.ShapeDtypeStruct((B, D), jnp.float32),
                  jax.ShapeDtypeStruct((B, D), jnp.float32)),
        mesh=mesh,
        scratch_types=(pltpu.VMEM((C, D), jnp.float32),
                       pltpu.VMEM((80,), jnp.int32),
                       pltpu.VMEM((GPW, D), jnp.float32),
                       pltpu.VMEM((GPW, D), jnp.float32),
                       pltpu.VMEM((GPW, D), jnp.float32)))


def _silu(v):
    return v * (1.0 / (1.0 + jnp.exp(-v)))


def _sage_tc(x_ref, s_ref, m_ref, cnt_ref, wl_ref, wr_ref, bl_ref, o_ref):
    s = s_ref[...]
    inv = 1.0 / jnp.maximum(cnt_ref[...], 1.0)
    agg = jnp.concatenate([m_ref[...], s, s * inv], axis=1)
    y = (jnp.dot(agg, wl_ref[...], preferred_element_type=jnp.float32)
         + jnp.dot(x_ref[...], wr_ref[...], preferred_element_type=jnp.float32)
         + bl_ref[...])
    o_ref[...] = _silu(y)


def _mlp_tc(h_ref, w1_ref, b1_ref, g1_ref, be1_ref, w2_ref, b2_ref, o_ref):
    t = jnp.dot(h_ref[...], w1_ref[...], preferred_element_type=jnp.float32) + b1_ref[...]
    t = _silu(t)
    mu = t.mean(-1, keepdims=True)
    var = ((t - mu) ** 2).mean(-1, keepdims=True)
    t = (t - mu) * lax.rsqrt(var + 1e-5) * g1_ref[...] + be1_ref[...]
    o_ref[...] = jnp.dot(t, w2_ref[...], preferred_element_type=jnp.float32) + b2_ref[...]


def _ro_tc(mn_ref, mx_ref, sm_ref, w1_ref, b1_ref, g1_ref, be1_ref, w2_ref, b2_ref, o_ref):
    out = jnp.concatenate([mn_ref[...], mx_ref[...], sm_ref[...]], axis=1)
    t = jnp.dot(out, w1_ref[...], preferred_element_type=jnp.float32) + b1_ref[...]
    t = _silu(t)
    mu = t.mean(-1, keepdims=True)
    var = ((t - mu) ** 2).mean(-1, keepdims=True)
    t = (t - mu) * lax.rsqrt(var + 1e-5) * g1_ref[...] + be1_ref[...]
    o_ref[...] = jnp.dot(t, w2_ref[...], preferred_element_type=jnp.float32) + b2_ref[...]


def _row_block(shape):
    return pl.BlockSpec(shape, lambda i: (i, 0))


def _whole(shape):
    return pl.BlockSpec(shape, lambda i: (0, 0))


_GRID = NPAD // R

_sage_call = pl.pallas_call(
    _sage_tc,
    grid=(_GRID,),
    in_specs=[_row_block((R, D)), _row_block((R, D)), _row_block((R, D)),
              _row_block((R, 1)),
              _whole((3 * D, D)), _whole((D, D)), _whole((1, D))],
    out_specs=_row_block((R, D)),
    out_shape=jax.ShapeDtypeStruct((NPAD, D), jnp.float32),
)

_mlp_call = pl.pallas_call(
    _mlp_tc,
    grid=(_GRID,),
    in_specs=[_row_block((R, D)),
              _whole((D, 4 * D)), _whole((1, 4 * D)), _whole((1, 4 * D)),
              _whole((1, 4 * D)), _whole((4 * D, D)), _whole((1, D))],
    out_specs=_row_block((R, D)),
    out_shape=jax.ShapeDtypeStruct((NPAD, D), jnp.float32),
)

_ro_call = pl.pallas_call(
    _ro_tc,
    in_specs=[pl.BlockSpec((B, D)), pl.BlockSpec((B, D)), pl.BlockSpec((B, D)),
              pl.BlockSpec((3 * D, 4 * D)), pl.BlockSpec((1, 4 * D)),
              pl.BlockSpec((1, 4 * D)), pl.BlockSpec((1, 4 * D)),
              pl.BlockSpec((4 * D, D)), pl.BlockSpec((1, D))],
    out_specs=pl.BlockSpec((B, D)),
    out_shape=jax.ShapeDtypeStruct((B, D), jnp.float32),
)


def kernel(x, edge_index, batch, params):
    src = edge_index[0]
    dst = edge_index[1]
    dst_s, src_s = lax.sort([dst, src], num_keys=1)
    counts = jnp.zeros((NPAD + NPW + 15,), jnp.int32).at[dst].add(1)
    rp = jnp.concatenate([jnp.zeros((1,), jnp.int32),
                          jnp.cumsum(counts, dtype=jnp.int32)])
    src_p = jnp.concatenate([src_s, jnp.zeros((EPAD - E,), jnp.int32)])
    dst_p = jnp.concatenate([dst_s, jnp.full((EPAD - E,), NPAD, jnp.int32)])
    bp = jnp.searchsorted(
        batch, jnp.arange(80, dtype=jnp.int32), side="left").astype(jnp.int32)

    h = jnp.zeros((NPAD, D), jnp.float32).at[:N].set(x)
    for p in params["convs"]:
        ssum, smax, scnt = _agg_call()(h, src_p, dst_p, rp)
        h = _sage_call(h, ssum, smax, scnt.reshape(NPAD, 1),
                       p["Wl"], p["Wr"], p["bl"].reshape(1, D))

    m = params["mlp"]
    h = _mlp_call(h, m["W1"], m["b1"].reshape(1, 4 * D), m["g1"].reshape(1, 4 * D),
                  m["be1"].reshape(1, 4 * D), m["W2"], m["b2"].reshape(1, D))

    psum, pmax, pmean = _pool_call()(h, bp)

    r = params["ro"]
    w2p = jnp.zeros((4 * D, D), jnp.float32).at[:, :2 * NOUT].set(r["W2"])
    b2p = jnp.zeros((1, D), jnp.float32).at[0, :2 * NOUT].set(r["b2"])
    out = _ro_call(pmean, pmax, psum,
                   r["W1"], r["b1"].reshape(1, 4 * D), r["g1"].reshape(1, 4 * D),
                   r["be1"].reshape(1, 4 * D), w2p, b2p)
    return out[:, :2 * NOUT]
